# baseline (speedup 1.0000x reference)
import functools

import jax
import jax.numpy as jnp
from jax import lax
from jax.experimental import pallas as pl
from jax.experimental.pallas import tpu as pltpu
from jax.experimental.pallas import tpu_sc as plsc

L = 16


@functools.lru_cache(maxsize=None)
def _make_gather_kernel(B, ZD, V, D, NC, NS):
    NW = NC * NS
    assert B % (8 * NW) == 0 and V % 2 == 0 and 2 * D == 128
    b_per_w = B // NW
    assert b_per_w % L == 0 and b_per_w <= 128

    mesh = plsc.VectorSubcoreMesh(
        core_axis_name="c", subcore_axis_name="s", num_cores=NC, num_subcores=NS
    )

    @functools.partial(
        pl.kernel,
        mesh=mesh,
        out_type=(
            jax.ShapeDtypeStruct((B, 2 * D), jnp.float32),
            jax.ShapeDtypeStruct((B,), jnp.int32),
        ),
        scratch_types=[
            pltpu.VMEM((b_per_w,), jnp.int32),
            pltpu.VMEM((ZD, b_per_w), jnp.int32),
            pltpu.VMEM((b_per_w,), jnp.int32),
            pltpu.VMEM((b_per_w,), jnp.int32),
            pltpu.VMEM((b_per_w, 2 * D), jnp.float32),
            pltpu.SemaphoreType.DMA,
        ],
    )
    def k(obs_hbm, zt_hbm, table2_hbm, pairs_hbm, seeds_hbm,
          obs_v, z_v, sv_v, idx_v, rows_v, sem):
        wid = lax.axis_index("s") * NC + lax.axis_index("c")
        base = wid * b_per_w
        pltpu.sync_copy(obs_hbm.at[pl.ds(base, b_per_w)], obs_v)
        pltpu.sync_copy(zt_hbm.at[:, pl.ds(base, b_per_w)], z_v)
        for g in range(b_per_w // L):
            acc = obs_v[pl.ds(g * L, L)]
            for j in range(ZD):
                bits = z_v[j, pl.ds(g * L, L)]
                acc = acc + bits * (1 << (ZD - 1 - j))
            s = jnp.where(acc >= V, acc - V, acc)
            sv_v[pl.ds(g * L, L)] = s
            idx_v[pl.ds(g * L, L)] = s >> 1
        pltpu.async_copy(table2_hbm.at[idx_v], rows_v, sem).wait()
        pltpu.sync_copy(rows_v, pairs_hbm.at[pl.ds(base, b_per_w)])
        pltpu.sync_copy(sv_v, seeds_hbm.at[pl.ds(base, b_per_w)])

    return k


def _select_body(seeds_ref, pairs_ref, out_ref):
    odd = (seeds_ref[...] & 1) == 1
    d = out_ref.shape[1]
    out_ref[...] = jnp.where(odd, pairs_ref[:, d:], pairs_ref[:, :d])


def kernel(obs_hash, z, seed_to_action):
    B, ZD = z.shape
    V, D = seed_to_action.shape
    info = plsc.get_sparse_core_info()
    k = _make_gather_kernel(B, ZD, V, D, info.num_cores, info.num_subcores)
    pairs, seeds = k(
        obs_hash.astype(jnp.int32),
        z.astype(jnp.int32).T,
        seed_to_action.reshape(V // 2, 2 * D),
    )
    return pl.pallas_call(
        _select_body,
        out_shape=jax.ShapeDtypeStruct((B, D), jnp.float32),
    )(seeds.reshape(B, 1), pairs)
